# baseline (speedup 1.0000x reference)
import functools

import jax
import jax.numpy as jnp
from jax import lax
from jax.experimental import pallas as pl
from jax.experimental.pallas import tpu as pltpu
from jax.experimental.pallas import tpu_sc as plsc

N = 10000
E = 320000
D = 128

NC = 2
NS = 16
EPT = E // (NC * NS)
CH = 80
NCHUNK = EPT // CH
DEPTH = 4
NP = 10240
RPT = NP // NS

_mesh = plsc.VectorSubcoreMesh(core_axis_name="c", subcore_axis_name="s")


@functools.partial(
    pl.kernel,
    mesh=_mesh,
    out_type=jax.ShapeDtypeStruct((NC, NP, D), jnp.float32),
    scratch_types=[
        pltpu.VMEM((DEPTH, 2, CH), jnp.int32),
        pltpu.VMEM((DEPTH, CH, D), jnp.float32),
        pltpu.VMEM_SHARED((NP, D), jnp.float32),
        [pltpu.SemaphoreType.DMA] * DEPTH,
        [pltpu.SemaphoreType.DMA] * DEPTH,
    ],
)
def _segsum(h_hbm, idx_hbm, zeros_hbm, out_hbm, ibuf, gbuf, acc, isems, gsems):
    c = lax.axis_index("c")
    s = lax.axis_index("s")
    pltpu.sync_copy(zeros_hbm.at[pl.ds(s * RPT, RPT)], acc.at[pl.ds(s * RPT, RPT)])
    plsc.subcore_barrier()

    def fire_idx(j, b):
        pltpu.async_copy(idx_hbm.at[c, s, j], ibuf.at[b], isems[b])

    def wait_idx(j, b):
        pltpu.make_async_copy(idx_hbm.at[c, s, j], ibuf.at[b], isems[b]).wait()

    def fire_gather(b):
        pltpu.async_copy(h_hbm.at[ibuf.at[b, 0]], gbuf.at[b], gsems[b])

    def wait_gather_scatter(b):
        pltpu.make_async_copy(h_hbm.at[ibuf.at[b, 0]], gbuf.at[b], gsems[b]).wait()
        pltpu.sync_copy(gbuf.at[b], acc.at[ibuf.at[b, 1]], add=True)

    for b in range(DEPTH):
        fire_idx(b, b)
    for b in range(DEPTH - 1):
        wait_idx(b, b)
        fire_gather(b)

    def rounds(i, carry):
        j0 = DEPTH * i
        for u in range(DEPTH):
            j = j0 + u
            b = u
            lead = j + DEPTH - 1
            bl = (b + DEPTH - 1) % DEPTH

            @pl.when(lead < NCHUNK)
            def _():
                wait_idx(lead, bl)
                fire_gather(bl)

            wait_gather_scatter(b)

            @pl.when(j + DEPTH < NCHUNK)
            def _():
                fire_idx(j + DEPTH, b)

        return carry

    lax.fori_loop(0, NCHUNK // DEPTH, rounds, 0)
    for j in range(NCHUNK - NCHUNK % DEPTH, NCHUNK):
        b = j % DEPTH
        lead = j + DEPTH - 1
        if lead < NCHUNK:
            wait_idx(lead, (b + DEPTH - 1) % DEPTH)
            fire_gather((b + DEPTH - 1) % DEPTH)
        wait_gather_scatter(b)
    plsc.subcore_barrier()
    pltpu.sync_copy(acc.at[pl.ds(s * RPT, RPT)], out_hbm.at[c].at[pl.ds(s * RPT, RPT)])


def _proj_body(act, h_ref, s_ref, wt_ref, b_ref, o_ref):
    hs = h_ref[...] * (s_ref[0] + s_ref[1])
    y = jnp.dot(hs, wt_ref[...], preferred_element_type=jnp.float32) + b_ref[...]
    if act:
        y = y * lax.logistic(y)
    o_ref[...] = y


def _project(h, s2, wt, b2d, act):
    bn = 1000
    return pl.pallas_call(
        functools.partial(_proj_body, act),
        out_shape=jax.ShapeDtypeStruct((N, D), jnp.float32),
        grid=(N // bn,),
        in_specs=[
            pl.BlockSpec((bn, D), lambda i: (i, 0)),
            pl.BlockSpec((NC, bn, D), lambda i: (0, i, 0)),
            pl.BlockSpec((D, D), lambda i: (0, 0)),
            pl.BlockSpec((1, D), lambda i: (0, 0)),
        ],
        out_specs=pl.BlockSpec((bn, D), lambda i: (i, 0)),
    )(h, s2, wt, b2d)


def kernel(x, edge_index, W1, b1, W2, b2, W3, b3):
    ei = edge_index.reshape(2, NC, NS, NCHUNK, CH)
    idx = jnp.stack([ei[1], ei[0]], axis=3)
    zeros = jnp.zeros((NP, D), jnp.float32)
    h = x
    for W, b, act in ((W1, b1, True), (W2, b2, True), (W3, b3, False)):
        s2 = _segsum(h, idx, zeros)
        h = _project(h, s2, W.T, b.reshape(1, D), act)
    return h


# speedup vs baseline: 1.1520x; 1.1520x over previous
import functools

import jax
import jax.numpy as jnp
from jax import lax
from jax.experimental import pallas as pl
from jax.experimental.pallas import tpu as pltpu
from jax.experimental.pallas import tpu_sc as plsc

N = 10000
E = 320000
D = 128

NC = 2
NS = 16
EPT = E // (NC * NS)
CH = 125
NCHUNK = EPT // CH
DEPTH = 3
NP = 10112
RPT = NP // NS

_mesh = plsc.VectorSubcoreMesh(core_axis_name="c", subcore_axis_name="s")


@functools.partial(
    pl.kernel,
    mesh=_mesh,
    out_type=jax.ShapeDtypeStruct((NC, NP, D), jnp.float32),
    scratch_types=[
        pltpu.VMEM((DEPTH, 2, CH), jnp.int32),
        pltpu.VMEM((DEPTH, CH, D), jnp.float32),
        pltpu.VMEM_SHARED((NP, D), jnp.float32),
        [pltpu.SemaphoreType.DMA] * DEPTH,
        [pltpu.SemaphoreType.DMA] * DEPTH,
    ],
)
def _segsum(h_hbm, idx_hbm, zeros_hbm, out_hbm, ibuf, gbuf, acc, isems, gsems):
    c = lax.axis_index("c")
    s = lax.axis_index("s")
    pltpu.sync_copy(zeros_hbm.at[pl.ds(s * RPT, RPT)], acc.at[pl.ds(s * RPT, RPT)])
    plsc.subcore_barrier()

    def fire_idx(j, b):
        pltpu.async_copy(idx_hbm.at[c, s, j], ibuf.at[b], isems[b])

    def wait_idx(j, b):
        pltpu.make_async_copy(idx_hbm.at[c, s, j], ibuf.at[b], isems[b]).wait()

    def fire_gather(b):
        pltpu.async_copy(h_hbm.at[ibuf.at[b, 0]], gbuf.at[b], gsems[b])

    def wait_gather_scatter(b):
        pltpu.make_async_copy(h_hbm.at[ibuf.at[b, 0]], gbuf.at[b], gsems[b]).wait()
        pltpu.sync_copy(gbuf.at[b], acc.at[ibuf.at[b, 1]], add=True)

    for b in range(DEPTH):
        fire_idx(b, b)
    for b in range(DEPTH - 1):
        wait_idx(b, b)
        fire_gather(b)

    def rounds(i, carry):
        j0 = DEPTH * i
        for u in range(DEPTH):
            j = j0 + u
            b = u
            lead = j + DEPTH - 1
            bl = (b + DEPTH - 1) % DEPTH

            @pl.when(lead < NCHUNK)
            def _():
                wait_idx(lead, bl)
                fire_gather(bl)

            wait_gather_scatter(b)

            @pl.when(j + DEPTH < NCHUNK)
            def _():
                fire_idx(j + DEPTH, b)

        return carry

    lax.fori_loop(0, NCHUNK // DEPTH, rounds, 0)
    for j in range(NCHUNK - NCHUNK % DEPTH, NCHUNK):
        b = j % DEPTH
        lead = j + DEPTH - 1
        if lead < NCHUNK:
            wait_idx(lead, (b + DEPTH - 1) % DEPTH)
            fire_gather((b + DEPTH - 1) % DEPTH)
        wait_gather_scatter(b)
    plsc.subcore_barrier()
    pltpu.sync_copy(acc.at[pl.ds(s * RPT, RPT)], out_hbm.at[c].at[pl.ds(s * RPT, RPT)])


def _proj_body(act, h_ref, s_ref, wt_ref, b_ref, o_ref):
    hs = h_ref[...] * (s_ref[0] + s_ref[1])
    y = jnp.dot(hs, wt_ref[...], preferred_element_type=jnp.float32) + b_ref[...]
    if act:
        y = y * lax.logistic(y)
    o_ref[...] = y


def _project(h, s2, wt, b2d, act):
    bn = 1000
    return pl.pallas_call(
        functools.partial(_proj_body, act),
        out_shape=jax.ShapeDtypeStruct((N, D), jnp.float32),
        grid=(N // bn,),
        in_specs=[
            pl.BlockSpec((bn, D), lambda i: (i, 0)),
            pl.BlockSpec((NC, bn, D), lambda i: (0, i, 0)),
            pl.BlockSpec((D, D), lambda i: (0, 0)),
            pl.BlockSpec((1, D), lambda i: (0, 0)),
        ],
        out_specs=pl.BlockSpec((bn, D), lambda i: (i, 0)),
    )(h, s2, wt, b2d)


def kernel(x, edge_index, W1, b1, W2, b2, W3, b3):
    ei = edge_index.reshape(2, NC, NS, NCHUNK, CH)
    idx = jnp.stack([ei[1], ei[0]], axis=3)
    zeros = jnp.zeros((NP, D), jnp.float32)
    h = x
    for W, b, act in ((W1, b1, True), (W2, b2, True), (W3, b3, False)):
        s2 = _segsum(h, idx, zeros)
        h = _project(h, s2, W.T, b.reshape(1, D), act)
    return h


# speedup vs baseline: 1.2475x; 1.0829x over previous
import functools

import jax
import jax.numpy as jnp
from jax import lax
from jax.experimental import pallas as pl
from jax.experimental.pallas import tpu as pltpu
from jax.experimental.pallas import tpu_sc as plsc

N = 10000
E = 320000
D = 128

NC = 2
NS = 16
EPT = E // (NC * NS)
CH = 125
NCHUNK = EPT // CH
DEPTH = 3
NP = 10112
RPT = NP // NS

_mesh = plsc.VectorSubcoreMesh(core_axis_name="c", subcore_axis_name="s")


@functools.partial(
    pl.kernel,
    mesh=_mesh,
    out_type=jax.ShapeDtypeStruct((NC, NP, D), jnp.float32),
    scratch_types=[
        pltpu.VMEM((DEPTH, 2, CH), jnp.int32),
        pltpu.VMEM((DEPTH, CH, D), jnp.float32),
        pltpu.VMEM_SHARED((NP, D), jnp.float32),
        [pltpu.SemaphoreType.DMA] * DEPTH,
        [pltpu.SemaphoreType.DMA] * DEPTH,
        [pltpu.SemaphoreType.DMA] * DEPTH,
    ],
)
def _segsum(h_hbm, idx_hbm, zeros_hbm, out_hbm, ibuf, gbuf, acc,
            isems, gsems, ssems):
    c = lax.axis_index("c")
    s = lax.axis_index("s")
    pltpu.sync_copy(zeros_hbm.at[pl.ds(s * RPT, RPT)], acc.at[pl.ds(s * RPT, RPT)])
    plsc.subcore_barrier()

    def fire_idx(j, b):
        pltpu.async_copy(idx_hbm.at[c, s, j], ibuf.at[b], isems[b])

    def wait_idx(j, b):
        pltpu.make_async_copy(idx_hbm.at[c, s, j], ibuf.at[b], isems[b]).wait()

    def fire_gather(b):
        pltpu.async_copy(h_hbm.at[ibuf.at[b, 0]], gbuf.at[b], gsems[b])

    def wait_gather(b):
        pltpu.make_async_copy(h_hbm.at[ibuf.at[b, 0]], gbuf.at[b], gsems[b]).wait()

    def fire_scatter(b):
        pltpu.async_copy(gbuf.at[b], acc.at[ibuf.at[b, 1]], ssems[b], add=True)

    def wait_scatter(b):
        pltpu.make_async_copy(gbuf.at[b], acc.at[ibuf.at[b, 1]], ssems[b]).wait()

    for b in range(DEPTH - 1):
        fire_idx(b, b)
    wait_idx(0, 0)
    fire_gather(0)

    def step(j, b):
        bn = (b + 1) % DEPTH
        bp = (b + DEPTH - 1) % DEPTH

        @pl.when(j + 1 < NCHUNK)
        def _():
            wait_idx(j + 1, bn)
            fire_gather(bn)

        wait_gather(b)
        fire_scatter(b)

        @pl.when(j - 1 >= 0)
        def _():
            wait_scatter(bp)

        @pl.when(j + DEPTH - 1 < NCHUNK)
        def _():
            fire_idx(j + DEPTH - 1, bp)

    def rounds(i, carry):
        j0 = DEPTH * i
        for u in range(DEPTH):
            step(j0 + u, u)
        return carry

    lax.fori_loop(0, NCHUNK // DEPTH, rounds, 0)
    for j in range(NCHUNK - NCHUNK % DEPTH, NCHUNK):
        step(j, j % DEPTH)
    wait_scatter((NCHUNK - 1) % DEPTH)
    plsc.subcore_barrier()
    pltpu.sync_copy(acc.at[pl.ds(s * RPT, RPT)], out_hbm.at[c].at[pl.ds(s * RPT, RPT)])


def _proj_body(act, h_ref, s_ref, wt_ref, b_ref, o_ref):
    hs = h_ref[...] * (s_ref[0] + s_ref[1])
    y = jnp.dot(hs, wt_ref[...], preferred_element_type=jnp.float32) + b_ref[...]
    if act:
        y = y * lax.logistic(y)
    o_ref[...] = y


def _project(h, s2, wt, b2d, act):
    bn = 1000
    return pl.pallas_call(
        functools.partial(_proj_body, act),
        out_shape=jax.ShapeDtypeStruct((N, D), jnp.float32),
        grid=(N // bn,),
        in_specs=[
            pl.BlockSpec((bn, D), lambda i: (i, 0)),
            pl.BlockSpec((NC, bn, D), lambda i: (0, i, 0)),
            pl.BlockSpec((D, D), lambda i: (0, 0)),
            pl.BlockSpec((1, D), lambda i: (0, 0)),
        ],
        out_specs=pl.BlockSpec((bn, D), lambda i: (i, 0)),
    )(h, s2, wt, b2d)


def kernel(x, edge_index, W1, b1, W2, b2, W3, b3):
    ei = edge_index.reshape(2, NC, NS, NCHUNK, CH)
    idx = jnp.stack([ei[1], ei[0]], axis=3)
    zeros = jnp.zeros((NP, D), jnp.float32)
    h = x
    for W, b, act in ((W1, b1, True), (W2, b2, True), (W3, b3, False)):
        s2 = _segsum(h, idx, zeros)
        h = _project(h, s2, W.T, b.reshape(1, D), act)
    return h


# speedup vs baseline: 1.2697x; 1.0178x over previous
import functools

import jax
import jax.numpy as jnp
from jax import lax
from jax.experimental import pallas as pl
from jax.experimental.pallas import tpu as pltpu
from jax.experimental.pallas import tpu_sc as plsc

N = 10000
E = 320000
D = 128

NC = 2
NS = 16
EPT = E // (NC * NS)
CH = 125
NCHUNK = EPT // CH
DEPTH = 3
NP = 10112
RPT = NP // NS

_mesh = plsc.VectorSubcoreMesh(core_axis_name="c", subcore_axis_name="s")


@functools.partial(
    pl.kernel,
    mesh=_mesh,
    out_type=jax.ShapeDtypeStruct((NC, NP, D), jnp.float32),
    scratch_types=[
        pltpu.VMEM((DEPTH, 2, CH), jnp.int32),
        pltpu.VMEM((DEPTH, CH, D), jnp.float32),
        pltpu.VMEM_SHARED((NP, D), jnp.float32),
        [pltpu.SemaphoreType.DMA] * DEPTH,
        [pltpu.SemaphoreType.DMA] * DEPTH,
        [pltpu.SemaphoreType.DMA] * DEPTH,
        pltpu.SemaphoreType.DMA,
    ],
)
def _segsum(h_hbm, idx_hbm, zeros_hbm, out_hbm, ibuf, gbuf, acc,
            isems, gsems, ssems, zsem):
    c = lax.axis_index("c")
    s = lax.axis_index("s")
    zcopy = pltpu.async_copy(zeros_hbm.at[pl.ds(s * RPT, RPT)],
                             acc.at[pl.ds(s * RPT, RPT)], zsem)

    def fire_idx(j, b):
        pltpu.async_copy(idx_hbm.at[c, s, j], ibuf.at[b], isems[b])

    def wait_idx(j, b):
        pltpu.make_async_copy(idx_hbm.at[c, s, j], ibuf.at[b], isems[b]).wait()

    def fire_gather(b):
        pltpu.async_copy(h_hbm.at[ibuf.at[b, 0]], gbuf.at[b], gsems[b])

    def wait_gather(b):
        pltpu.make_async_copy(h_hbm.at[ibuf.at[b, 0]], gbuf.at[b], gsems[b]).wait()

    def fire_scatter(b):
        pltpu.async_copy(gbuf.at[b], acc.at[ibuf.at[b, 1]], ssems[b], add=True)

    def wait_scatter(b):
        pltpu.make_async_copy(gbuf.at[b], acc.at[ibuf.at[b, 1]], ssems[b]).wait()

    for b in range(DEPTH - 1):
        fire_idx(b, b)
    wait_idx(0, 0)
    fire_gather(0)
    zcopy.wait()
    plsc.subcore_barrier()

    def step(j, b):
        bn = (b + 1) % DEPTH
        bp = (b + DEPTH - 1) % DEPTH

        @pl.when(j + 1 < NCHUNK)
        def _():
            wait_idx(j + 1, bn)
            fire_gather(bn)

        wait_gather(b)
        fire_scatter(b)

        @pl.when(j - 1 >= 0)
        def _():
            wait_scatter(bp)

        @pl.when(j + DEPTH - 1 < NCHUNK)
        def _():
            fire_idx(j + DEPTH - 1, bp)

    def rounds(i, carry):
        j0 = DEPTH * i
        for u in range(DEPTH):
            step(j0 + u, u)
        return carry

    lax.fori_loop(0, NCHUNK // DEPTH, rounds, 0)
    for j in range(NCHUNK - NCHUNK % DEPTH, NCHUNK):
        step(j, j % DEPTH)
    wait_scatter((NCHUNK - 1) % DEPTH)
    plsc.subcore_barrier()
    pltpu.sync_copy(acc.at[pl.ds(s * RPT, RPT)], out_hbm.at[c].at[pl.ds(s * RPT, RPT)])


def _proj_body(act, h_ref, s_ref, wt_ref, b_ref, o_ref):
    hs = h_ref[...] * (s_ref[0] + s_ref[1])
    y = jnp.dot(hs, wt_ref[...], preferred_element_type=jnp.float32) + b_ref[...]
    if act:
        y = y * lax.logistic(y)
    o_ref[...] = y


def _project(h, s2, wt, b2d, act):
    bn = 1000
    return pl.pallas_call(
        functools.partial(_proj_body, act),
        out_shape=jax.ShapeDtypeStruct((N, D), jnp.float32),
        grid=(N // bn,),
        in_specs=[
            pl.BlockSpec((bn, D), lambda i: (i, 0)),
            pl.BlockSpec((NC, bn, D), lambda i: (0, i, 0)),
            pl.BlockSpec((D, D), lambda i: (0, 0)),
            pl.BlockSpec((1, D), lambda i: (0, 0)),
        ],
        out_specs=pl.BlockSpec((bn, D), lambda i: (i, 0)),
    )(h, s2, wt, b2d)


def kernel(x, edge_index, W1, b1, W2, b2, W3, b3):
    ei = edge_index.reshape(2, NC, NS, NCHUNK, CH)
    idx = jnp.stack([ei[1], ei[0]], axis=3)
    zeros = jnp.zeros((NP, D), jnp.float32)
    h = x
    for W, b, act in ((W1, b1, True), (W2, b2, True), (W3, b3, False)):
        s2 = _segsum(h, idx, zeros)
        h = _project(h, s2, W.T, b.reshape(1, D), act)
    return h


# speedup vs baseline: 1.3813x; 1.0879x over previous
import functools

import jax
import jax.numpy as jnp
from jax import lax
from jax.experimental import pallas as pl
from jax.experimental.pallas import tpu as pltpu
from jax.experimental.pallas import tpu_sc as plsc

N = 10000
E = 320000
D = 128

NC = 2
NS = 16
EPT = E // (NC * NS)
CH = 125
NCHUNK = EPT // CH
DEPTH = 3
IB = 4
UNROLL = 12
NP = 10112
RPT = NP // NS

_mesh = plsc.VectorSubcoreMesh(core_axis_name="c", subcore_axis_name="s")


@functools.partial(
    pl.kernel,
    mesh=_mesh,
    out_type=jax.ShapeDtypeStruct((NC, NP, D), jnp.float32),
    scratch_types=[
        pltpu.VMEM((IB, 2, CH), jnp.int32),
        pltpu.VMEM((DEPTH, CH, D), jnp.float32),
        pltpu.VMEM_SHARED((NP, D), jnp.float32),
        [pltpu.SemaphoreType.DMA] * IB,
        [pltpu.SemaphoreType.DMA] * DEPTH,
        [pltpu.SemaphoreType.DMA] * DEPTH,
        pltpu.SemaphoreType.DMA,
    ],
)
def _segsum(h_hbm, idx_hbm, zeros_hbm, out_hbm, ibuf, gbuf, acc,
            isems, gsems, ssems, zsem):
    c = lax.axis_index("c")
    s = lax.axis_index("s")
    zcopy = pltpu.async_copy(zeros_hbm.at[pl.ds(s * RPT, RPT)],
                             acc.at[pl.ds(s * RPT, RPT)], zsem)

    def fire_idx(j, bi):
        pltpu.async_copy(idx_hbm.at[c, s, j], ibuf.at[bi], isems[bi])

    def wait_idx(j, bi):
        pltpu.make_async_copy(idx_hbm.at[c, s, j], ibuf.at[bi], isems[bi]).wait()

    def fire_gather(bg, bi):
        pltpu.async_copy(h_hbm.at[ibuf.at[bi, 0]], gbuf.at[bg], gsems[bg])

    def wait_gather(bg, bi):
        pltpu.make_async_copy(h_hbm.at[ibuf.at[bi, 0]], gbuf.at[bg],
                              gsems[bg]).wait()

    def fire_scatter(bg, bi):
        pltpu.async_copy(gbuf.at[bg], acc.at[ibuf.at[bi, 1]], ssems[bg],
                         add=True)

    def wait_scatter(bg, bi):
        pltpu.make_async_copy(gbuf.at[bg], acc.at[ibuf.at[bi, 1]],
                              ssems[bg]).wait()

    for b in range(IB - 1):
        fire_idx(b, b)
    wait_idx(0, 0)
    fire_gather(0, 0)
    wait_idx(1, 1)
    fire_gather(1, 1)
    zcopy.wait()
    plsc.subcore_barrier()

    def step(j, bg, bi):
        @pl.when(j - 1 >= 0)
        def _():
            wait_scatter((bg + DEPTH - 1) % DEPTH, (bi + IB - 1) % IB)

        @pl.when(j + 2 < NCHUNK)
        def _():
            wait_idx(j + 2, (bi + 2) % IB)
            fire_gather((bg + 2) % DEPTH, (bi + 2) % IB)

        wait_gather(bg, bi)
        fire_scatter(bg, bi)

        @pl.when(j + 3 < NCHUNK)
        def _():
            fire_idx(j + 3, (bi + 3) % IB)

    def rounds(i, carry):
        j0 = UNROLL * i
        for u in range(UNROLL):
            step(j0 + u, u % DEPTH, u % IB)
        return carry

    lax.fori_loop(0, NCHUNK // UNROLL, rounds, 0)
    for j in range(NCHUNK - NCHUNK % UNROLL, NCHUNK):
        step(j, j % DEPTH, j % IB)
    wait_scatter((NCHUNK - 1) % DEPTH, (NCHUNK - 1) % IB)
    plsc.subcore_barrier()
    pltpu.sync_copy(acc.at[pl.ds(s * RPT, RPT)], out_hbm.at[c].at[pl.ds(s * RPT, RPT)])


def _proj_body(act, h_ref, s_ref, wt_ref, b_ref, o_ref):
    hs = h_ref[...] * (s_ref[0] + s_ref[1])
    y = jnp.dot(hs, wt_ref[...], preferred_element_type=jnp.float32) + b_ref[...]
    if act:
        y = y * lax.logistic(y)
    o_ref[...] = y


def _project(h, s2, wt, b2d, act):
    bn = 1000
    return pl.pallas_call(
        functools.partial(_proj_body, act),
        out_shape=jax.ShapeDtypeStruct((N, D), jnp.float32),
        grid=(N // bn,),
        in_specs=[
            pl.BlockSpec((bn, D), lambda i: (i, 0)),
            pl.BlockSpec((NC, bn, D), lambda i: (0, i, 0)),
            pl.BlockSpec((D, D), lambda i: (0, 0)),
            pl.BlockSpec((1, D), lambda i: (0, 0)),
        ],
        out_specs=pl.BlockSpec((bn, D), lambda i: (i, 0)),
    )(h, s2, wt, b2d)


def kernel(x, edge_index, W1, b1, W2, b2, W3, b3):
    ei = edge_index.reshape(2, NC, NS, NCHUNK, CH)
    idx = jnp.stack([ei[1], ei[0]], axis=3)
    zeros = jnp.zeros((NP, D), jnp.float32)
    h = x
    for W, b, act in ((W1, b1, True), (W2, b2, True), (W3, b3, False)):
        s2 = _segsum(h, idx, zeros)
        h = _project(h, s2, W.T, b.reshape(1, D), act)
    return h
